# baseline (speedup 1.0000x reference)
import jax
import jax.numpy as jnp
from jax.experimental import pallas as pl
from jax.experimental.pallas import tpu as pltpu

_N = 10000
_NFEAT = 128
_NHID = 16
_NCLASS = 8
_BN = 400
_NB = _N // _BN
_CW = 1024
_NBAND = 10
_SRC = [min(_CW * k, _N - _CW) for k in range(_NBAND)]
_UB = [min(_CW * (k + 1), _N) for k in range(_NBAND)]
_IMAX = [max(i for i in range(_NB) if _BN * i < _UB[k])
         for k in range(_NBAND)]
_CHUNKS = [(_CW * k, _CW) for k in range(9)] + [(9216, _N - 9216)]


def _s1_body(x_ref, w1_ref, s1_ref):
    s1_ref[...] = jnp.dot(x_ref[...], w1_ref[...],
                          preferred_element_type=jnp.float32
                          ).astype(jnp.bfloat16)


def _sweep1_body(adj_ref, s1_ref, b1_ref, w2_ref,
                 out_ref, s2_ref, *band_refs):
    i = pl.program_id(0)

    @pl.when(i == 0)
    def _init():
        s2_ref[...] = jnp.zeros((_N, _NCLASS), jnp.bfloat16)

    p = jnp.zeros((_BN, _NHID), jnp.float32)
    o = jnp.zeros((_BN, _NCLASS), jnp.float32)
    for k, (lo, w) in enumerate(_CHUNKS):
        blk = adj_ref[:, lo:lo + w].astype(jnp.bfloat16)
        p = p + jnp.dot(blk, s1_ref[lo:lo + w, :],
                        preferred_element_type=jnp.float32)
        o = o + jnp.dot(blk, s2_ref[lo:lo + w, :],
                        preferred_element_type=jnp.float32)
        if k < 9:
            @pl.when(i <= _IMAX[k])
            def _write_band(k=k, blk=blk):
                band_refs[k][...] = blk

    @pl.when(i <= _IMAX[9])
    def _write_band9():
        band_refs[9][...] = adj_ref[:, _SRC[9]:_SRC[9] + _CW
                                    ].astype(jnp.bfloat16)

    h = jnp.maximum(p + b1_ref[...], 0.0)
    s2_ref[pl.ds(i * _BN, _BN), :] = jnp.dot(
        h, w2_ref[...], preferred_element_type=jnp.float32
        ).astype(jnp.bfloat16)
    out_ref[...] = o


def _sweep2_body(s2_ref, outp_ref, b2_ref, *band_refs_and_out):
    band_refs = band_refs_and_out[:_NBAND]
    out_ref = band_refs_and_out[_NBAND]
    i = pl.program_id(0)
    acc = outp_ref[...] + b2_ref[...]
    for k in range(_NBAND):
        s2s = s2_ref[pl.ds(_SRC[k], _CW), :]
        col = jax.lax.broadcasted_iota(jnp.int32, (_CW, _NCLASS), 0) + _SRC[k]
        lo = jnp.maximum(i * _BN, _CW * k)
        s2m = jnp.where(col >= lo, s2s, jnp.bfloat16(0))
        acc = acc + jnp.dot(band_refs[k][...], s2m,
                            preferred_element_type=jnp.float32)
    out_ref[...] = acc


@jax.jit
def kernel(x, adj, W1, b1, W2, b2):
    const = lambda i: (0, 0)
    row = lambda i: (i, 0)

    s1 = pl.pallas_call(
        _s1_body,
        out_shape=jax.ShapeDtypeStruct((_N, _NHID), jnp.bfloat16),
    )(x, W1)

    out_part, s2, *bands = pl.pallas_call(
        _sweep1_body,
        grid=(_NB,),
        in_specs=[
            pl.BlockSpec((_BN, _N), row),
            pl.BlockSpec((_N, _NHID), const),
            pl.BlockSpec((1, _NHID), const),
            pl.BlockSpec((_NHID, _NCLASS), const),
        ],
        out_specs=[
            pl.BlockSpec((_BN, _NCLASS), row),
            pl.BlockSpec((_N, _NCLASS), const),
        ] + [
            pl.BlockSpec((_BN, _CW),
                         lambda i, k=k: (jnp.minimum(i, _IMAX[k]), 0))
            for k in range(_NBAND)
        ],
        out_shape=[
            jax.ShapeDtypeStruct((_N, _NCLASS), jnp.float32),
            jax.ShapeDtypeStruct((_N, _NCLASS), jnp.bfloat16),
        ] + [
            jax.ShapeDtypeStruct((_BN * (_IMAX[k] + 1), _CW), jnp.bfloat16)
            for k in range(_NBAND)
        ],
        compiler_params=pltpu.CompilerParams(
            dimension_semantics=("arbitrary",),
        ),
    )(adj, s1, b1.reshape(1, _NHID), W2)

    out = pl.pallas_call(
        _sweep2_body,
        grid=(_NB,),
        in_specs=[
            pl.BlockSpec((_N, _NCLASS), const),
            pl.BlockSpec((_BN, _NCLASS), row),
            pl.BlockSpec((1, _NCLASS), const),
        ] + [
            pl.BlockSpec((_BN, _CW),
                         lambda i, k=k: (jnp.minimum(i, _IMAX[k]), 0))
            for k in range(_NBAND)
        ],
        out_specs=pl.BlockSpec((_BN, _NCLASS), row),
        out_shape=jax.ShapeDtypeStruct((_N, _NCLASS), jnp.float32),
        compiler_params=pltpu.CompilerParams(
            dimension_semantics=("arbitrary",),
        ),
    )(s2, out_part, b2.reshape(1, _NCLASS), *bands)
    return out
